# baseline (speedup 1.0000x reference)
import functools

import jax
import jax.numpy as jnp
from jax.experimental import pallas as pl
from jax.experimental.pallas import tpu as pltpu
from jax.experimental.pallas import tpu_sc as plsc

E = 8
K = 2
R = 16
DIN = 2048
DOUT = 2048
ER = E * R
SCALING = 32.0 / 16.0

N_TOK = 4096
_NC = 2
_NS = 16
_NW = _NC * _NS
_TOKS_PER_W = N_TOK // _NW
_GROUPS = _TOKS_PER_W // 16


def _expand_eye_matrix():
    col = jax.lax.broadcasted_iota(jnp.int32, (E, ER + E), 1)
    row = jax.lax.broadcasted_iota(jnp.int32, (E, ER + E), 0)
    expand = jnp.logical_and(col < ER, col // R == row)
    eye = col - ER == row
    return jnp.logical_or(expand, eye).astype(jnp.float32)


def _haux_kernel(x_ref, aw_ref, h_ref, gate_ref, gatet_ref):
    xt = x_ref[...]
    haux = jax.lax.dot_general(
        xt, aw_ref[...], (((1,), (1,)), ((), ())),
        preferred_element_type=jnp.float32)
    h_ref[...] = haux[:, :ER]
    gate_ref[...] = haux[:, ER:ER + E]
    gatet_ref[...] = jax.lax.dot_general(
        aw_ref[ER:ER + E, :], xt, (((1,), (1,)), ((), ())),
        preferred_element_type=jnp.float32)


def _sc_routing_kernel(gatet_hbm, rwt_hbm, g_v, o_v):
    wid = jax.lax.axis_index("s") * _NC + jax.lax.axis_index("c")
    base = wid * _TOKS_PER_W
    pltpu.sync_copy(gatet_hbm.at[:, pl.ds(base, _TOKS_PER_W)], g_v)
    for grp in range(_GROUPS):
        sl = pl.ds(grp * 16, 16)
        g = [g_v[e, sl] for e in range(E)]
        m = g[0]
        for e in range(1, E):
            m = jnp.maximum(m, g[e])
        p = [jnp.exp(g[e] - m) for e in range(E)]
        s = p[0]
        for e in range(1, E):
            s = s + p[e]
        probs = [p[e] / s for e in range(E)]
        m1 = probs[0]
        for e in range(1, E):
            m1 = jnp.maximum(m1, probs[e])
        i1 = jnp.full((16,), E, jnp.int32)
        for e in range(E - 1, -1, -1):
            i1 = jnp.where(probs[e] == m1, e, i1)
        p2 = [jnp.where(i1 == e, jnp.float32(-1.0), probs[e])
              for e in range(E)]
        m2 = p2[0]
        for e in range(1, E):
            m2 = jnp.maximum(m2, p2[e])
        i2 = jnp.full((16,), E, jnp.int32)
        for e in range(E - 1, -1, -1):
            i2 = jnp.where(p2[e] == m2, e, i2)
        denom = m1 + m2 + jnp.float32(1e-9)
        r1 = m1 / denom
        r2 = m2 / denom
        zero = jnp.zeros((16,), jnp.float32)
        for e in range(E):
            o_v[e, sl] = (jnp.where(i1 == e, r1, zero)
                          + jnp.where(i2 == e, r2, zero))
    pltpu.sync_copy(o_v, rwt_hbm.at[:, pl.ds(base, _TOKS_PER_W)])


_sc_routing = functools.partial(
    pl.kernel,
    out_type=jax.ShapeDtypeStruct((E, N_TOK), jnp.float32),
    mesh=plsc.VectorSubcoreMesh(core_axis_name="c", subcore_axis_name="s"),
    scratch_types=[
        pltpu.VMEM((E, _TOKS_PER_W), jnp.float32),
        pltpu.VMEM((E, _TOKS_PER_W), jnp.float32),
    ],
)(_sc_routing_kernel)


def _main_kernel(x_ref, w_ref, b_ref, h_ref, rwt_ref, b3_ref,
                 out_ref, rw_ref):
    rwx = jax.lax.dot_general(
        rwt_ref[...], _expand_eye_matrix(), (((0,), (0,)), ((), ())),
        preferred_element_type=jnp.float32)
    rw_ref[...] = rwx[:, ER:ER + E]
    hp = h_ref[...] * rwx[:, :ER] * SCALING
    out = jax.lax.dot_general(
        x_ref[...], w_ref[...], (((1,), (1,)), ((), ())),
        preferred_element_type=jnp.float32)
    out += b_ref[...]
    out += jax.lax.dot_general(
        hp, b3_ref[...], (((1,), (0,)), ((), ())),
        preferred_element_type=jnp.float32)
    out_ref[...] = out


@jax.jit
def kernel(x, W, b, Wr, A, Bm):
    Bsz, S, _ = x.shape
    N = Bsz * S
    x_flat = x.reshape(N, DIN)
    A2 = A.reshape(ER, DIN)
    AW = jnp.concatenate([A2, Wr], axis=0)
    B3 = Bm.transpose(0, 2, 1).reshape(ER, DOUT)
    b2d = b.reshape(1, DOUT)

    TN = 512
    grid = (N // TN,)

    h, gate, gate_t = pl.pallas_call(
        _haux_kernel,
        grid=grid,
        in_specs=[
            pl.BlockSpec((TN, DIN), lambda i: (i, 0)),
            pl.BlockSpec((ER + E, DIN), lambda i: (0, 0)),
        ],
        out_specs=[
            pl.BlockSpec((TN, ER), lambda i: (i, 0)),
            pl.BlockSpec((TN, E), lambda i: (i, 0)),
            pl.BlockSpec((E, TN), lambda i: (0, i)),
        ],
        out_shape=[
            jax.ShapeDtypeStruct((N, ER), jnp.float32),
            jax.ShapeDtypeStruct((N, E), jnp.float32),
            jax.ShapeDtypeStruct((E, N), jnp.float32),
        ],
        compiler_params=pltpu.CompilerParams(
            dimension_semantics=("parallel",)),
    )(x_flat, AW)

    rw_t = _sc_routing(gate_t)

    out, rw = pl.pallas_call(
        _main_kernel,
        grid=grid,
        in_specs=[
            pl.BlockSpec((TN, DIN), lambda i: (i, 0)),
            pl.BlockSpec((DOUT, DIN), lambda i: (0, 0)),
            pl.BlockSpec((1, DOUT), lambda i: (0, 0)),
            pl.BlockSpec((TN, ER), lambda i: (i, 0)),
            pl.BlockSpec((E, TN), lambda i: (0, i)),
            pl.BlockSpec((ER, DOUT), lambda i: (0, 0)),
        ],
        out_specs=[
            pl.BlockSpec((TN, DOUT), lambda i: (i, 0)),
            pl.BlockSpec((TN, E), lambda i: (i, 0)),
        ],
        out_shape=[
            jax.ShapeDtypeStruct((N, DOUT), jnp.float32),
            jax.ShapeDtypeStruct((N, E), jnp.float32),
        ],
        compiler_params=pltpu.CompilerParams(
            dimension_semantics=("parallel",)),
    )(x_flat, W, b2d, h, rw_t, B3)

    return (out.reshape(Bsz, S, DOUT),
            rw.reshape(Bsz, S, E),
            gate.reshape(Bsz, S, E))


# speedup vs baseline: 1.3174x; 1.3174x over previous
import jax
import jax.numpy as jnp
from jax.experimental import pallas as pl
from jax.experimental.pallas import tpu as pltpu

E = 8
K = 2
R = 16
DIN = 2048
DOUT = 2048
ER = E * R
SCALING = 32.0 / 16.0


def _expand_matrix():
    col = jax.lax.broadcasted_iota(jnp.int32, (E, ER), 1)
    row = jax.lax.broadcasted_iota(jnp.int32, (E, ER), 0)
    return (col // R == row).astype(jnp.float32)


def _fused_kernel(x_ref, w_ref, b_ref, a2_ref, wr_ref, b3_ref,
                  out_ref, rw_ref, gate_ref):
    xt = x_ref[...]

    h = jax.lax.dot_general(
        xt, a2_ref[...], (((1,), (1,)), ((), ())),
        preferred_element_type=jnp.float32)
    gate = jax.lax.dot_general(
        xt, wr_ref[...], (((1,), (1,)), ((), ())),
        preferred_element_type=jnp.float32)
    gate_ref[...] = gate

    m = jnp.max(gate, axis=-1, keepdims=True)
    p = jnp.exp(gate - m)
    p = p / jnp.sum(p, axis=-1, keepdims=True)

    e_iota = jax.lax.broadcasted_iota(jnp.int32, p.shape, 1)
    m1 = jnp.max(p, axis=-1, keepdims=True)
    i1 = jnp.min(jnp.where(p == m1, e_iota, E), axis=-1, keepdims=True)
    sel1 = e_iota == i1
    p2 = jnp.where(sel1, -jnp.inf, p)
    m2 = jnp.max(p2, axis=-1, keepdims=True)
    i2 = jnp.min(jnp.where(p2 == m2, e_iota, E), axis=-1, keepdims=True)
    sel2 = e_iota == i2
    denom = m1 + m2 + 1e-9
    rw = (jnp.where(sel1, m1, 0.0) + jnp.where(sel2, m2, 0.0)) / denom
    rw_ref[...] = rw

    rw_exp = jax.lax.dot_general(
        rw, _expand_matrix(), (((1,), (0,)), ((), ())),
        preferred_element_type=jnp.float32)
    hp = h * rw_exp * SCALING

    out = jax.lax.dot_general(
        xt, w_ref[...], (((1,), (1,)), ((), ())),
        preferred_element_type=jnp.float32)
    out += b_ref[...]
    out += jax.lax.dot_general(
        hp, b3_ref[...], (((1,), (0,)), ((), ())),
        preferred_element_type=jnp.float32)
    out_ref[...] = out


@jax.jit
def kernel(x, W, b, Wr, A, Bm):
    Bsz, S, _ = x.shape
    N = Bsz * S
    x_flat = x.reshape(N, DIN)
    A2 = A.reshape(ER, DIN)
    B3 = Bm.transpose(0, 2, 1).reshape(ER, DOUT)
    b2d = b.reshape(1, DOUT)

    TN = 512
    grid = (N // TN,)

    out, rw, gate = pl.pallas_call(
        _fused_kernel,
        grid=grid,
        in_specs=[
            pl.BlockSpec((TN, DIN), lambda i: (i, 0)),
            pl.BlockSpec((DOUT, DIN), lambda i: (0, 0)),
            pl.BlockSpec((1, DOUT), lambda i: (0, 0)),
            pl.BlockSpec((ER, DIN), lambda i: (0, 0)),
            pl.BlockSpec((E, DIN), lambda i: (0, 0)),
            pl.BlockSpec((ER, DOUT), lambda i: (0, 0)),
        ],
        out_specs=[
            pl.BlockSpec((TN, DOUT), lambda i: (i, 0)),
            pl.BlockSpec((TN, E), lambda i: (i, 0)),
            pl.BlockSpec((TN, E), lambda i: (i, 0)),
        ],
        out_shape=[
            jax.ShapeDtypeStruct((N, DOUT), jnp.float32),
            jax.ShapeDtypeStruct((N, E), jnp.float32),
            jax.ShapeDtypeStruct((N, E), jnp.float32),
        ],
        compiler_params=pltpu.CompilerParams(
            dimension_semantics=("parallel",)),
    )(x_flat, W, b2d, A2, Wr, B3)

    return (out.reshape(Bsz, S, DOUT),
            rw.reshape(Bsz, S, E),
            gate.reshape(Bsz, S, E))
